# baseline (speedup 1.0000x reference)
import functools

import jax
import jax.numpy as jnp
from jax import lax
from jax.experimental import pallas as pl
from jax.experimental.pallas import tpu as pltpu
from jax.experimental.pallas import tpu_sc as plsc

NC = 2
NS = 16
NW = NC * NS
LANES = 16
C = 16
GI = 128


@functools.partial(jax.jit, static_argnames=("B", "D", "NEG"))
def _sc_dots(in_pair, ctx_pair, neg_pair, in_par, ctx_par, neg_par,
             tab_in, tab_out, *, B, D, NEG):
    BPW = B // NW
    NCHUNK = BPW // C
    CN = C * NEG
    KD = D // LANES
    D2 = 2 * D

    mesh = plsc.VectorSubcoreMesh(core_axis_name="c", subcore_axis_name="s",
                                  num_cores=NC, num_subcores=NS)

    nsl = [(o, min(GI, CN - o)) for o in range(0, CN, GI)]

    @functools.partial(
        pl.kernel,
        out_type=(
            jax.ShapeDtypeStruct((B,), jnp.float32),
            jax.ShapeDtypeStruct((B * NEG,), jnp.float32),
        ),
        mesh=mesh,
        compiler_params=pltpu.CompilerParams(needs_layout_passes=False,
                                             use_tc_tiling_on_sc=False),
        scratch_types=[
            pltpu.VMEM((C,), jnp.int32),
            pltpu.VMEM((C,), jnp.int32),
            pltpu.VMEM((C,), jnp.int32),
            pltpu.VMEM((C,), jnp.int32),
            pltpu.VMEM((CN,), jnp.int32),
            pltpu.VMEM((CN,), jnp.int32),
            pltpu.VMEM((C,), jnp.int32),
            pltpu.VMEM((C,), jnp.int32),
            pltpu.VMEM((C,), jnp.int32),
            pltpu.VMEM((C,), jnp.int32),
            pltpu.VMEM((CN,), jnp.int32),
            pltpu.VMEM((CN,), jnp.int32),
            pltpu.VMEM((C, D2), jnp.float32),
            pltpu.VMEM((C, D2), jnp.float32),
            pltpu.VMEM((C, D2), jnp.float32),
            pltpu.VMEM((C, D2), jnp.float32),
            pltpu.VMEM((CN, D2), jnp.float32),
            pltpu.VMEM((CN, D2), jnp.float32),
            pltpu.VMEM((C, D), jnp.float32),
            pltpu.VMEM((C,), jnp.float32),
            pltpu.VMEM((CN,), jnp.float32),
            pltpu.SemaphoreType.DMA,
            pltpu.SemaphoreType.DMA,
        ],
    )
    def k(in_pair_h, ctx_pair_h, neg_pair_h, in_par_h, ctx_par_h, neg_par_h,
          tab_in_h, tab_out_h, pos_h, neg_h,
          ii0, ii1, ci0, ci1, ni0, ni1, ip0, ip1, cp0, cp1, np0, np1,
          ir0, ir1, cr0, cr1, nr0, nr1, is_v, po_v, no_v, sem0, sem1):
        ii_v, ci_v, ni_v = (ii0, ii1), (ci0, ci1), (ni0, ni1)
        ip_v, cp_v, np_v = (ip0, ip1), (cp0, cp1), (np0, np1)
        ir_v, cr_v, nr_v = (ir0, ir1), (cr0, cr1), (nr0, nr1)
        sems = (sem0, sem1)

        wid = lax.axis_index("s") * NC + lax.axis_index("c")
        lane = lax.iota(jnp.int32, LANES)
        lane_masks = [lane == l for l in range(LANES)]

        def stage(cid, b):
            sem = sems[b]
            pltpu.sync_copy(in_pair_h.at[pl.ds(cid * C, C)], ii_v[b])
            pltpu.sync_copy(ctx_pair_h.at[pl.ds(cid * C, C)], ci_v[b])
            pltpu.sync_copy(neg_pair_h.at[pl.ds(cid * CN, CN)], ni_v[b])
            pltpu.sync_copy(in_par_h.at[pl.ds(cid * C, C)], ip_v[b])
            pltpu.sync_copy(ctx_par_h.at[pl.ds(cid * C, C)], cp_v[b])
            pltpu.sync_copy(neg_par_h.at[pl.ds(cid * CN, CN)], np_v[b])
            pltpu.make_async_copy(tab_in_h.at[ii_v[b]], ir_v[b], sem).start()
            pltpu.make_async_copy(tab_out_h.at[ci_v[b]], cr_v[b], sem).start()
            for o, n in nsl:
                pltpu.make_async_copy(
                    tab_out_h.at[ni_v[b].at[pl.ds(o, n)]],
                    nr_v[b].at[pl.ds(o, n)], sem).start()

        def wait(b):
            sem = sems[b]
            pltpu.make_async_copy(tab_in_h.at[pl.ds(0, C)],
                                  ir_v[b], sem).wait()
            pltpu.make_async_copy(tab_in_h.at[pl.ds(0, C)],
                                  cr_v[b], sem).wait()
            pltpu.make_async_copy(tab_in_h.at[pl.ds(0, CN)],
                                  nr_v[b], sem).wait()

        def dot(row_chunks, in_chunks):
            acc = in_chunks[0] * row_chunks[0]
            for kk in range(1, KD):
                acc = acc + in_chunks[kk] * row_chunks[kk]
            return jnp.sum(acc)

        def pack16(scalars):
            vec = jnp.zeros((LANES,), jnp.float32)
            for l in range(LANES):
                vec = jnp.where(lane_masks[l], scalars[l], vec)
            return vec

        def compute(cid, b):
            for h in range(C // LANES):
                ipar = ip_v[b][pl.ds(h * LANES, LANES)]
                cpar = cp_v[b][pl.ds(h * LANES, LANES)]
                scal = []
                for l in range(LANES):
                    r = h * LANES + l
                    ioff = ipar[l] * D
                    for kk in range(KD):
                        is_v[r, pl.ds(kk * LANES, LANES)] = (
                            ir_v[b][r, pl.ds(ioff + kk * LANES, LANES)])
                    ivs = [is_v[r, pl.ds(kk * LANES, LANES)]
                           for kk in range(KD)]
                    coff = cpar[l] * D
                    cvs = [cr_v[b][r, pl.ds(coff + kk * LANES, LANES)]
                           for kk in range(KD)]
                    scal.append(dot(cvs, ivs))
                po_v[pl.ds(h * LANES, LANES)] = pack16(scal)

            @pl.loop(0, CN // LANES)
            def _(v):
                npar = np_v[b][pl.ds(v * LANES, LANES)]
                scal = []
                for l in range(LANES):
                    q = v * LANES + l
                    r = q // NEG
                    noff = npar[l] * D
                    ivs = [is_v[r, pl.ds(kk * LANES, LANES)]
                           for kk in range(KD)]
                    nvs = [nr_v[b][q, pl.ds(noff + kk * LANES, LANES)]
                           for kk in range(KD)]
                    scal.append(dot(nvs, ivs))
                no_v[pl.ds(v * LANES, LANES)] = pack16(scal)

            pltpu.sync_copy(po_v, pos_h.at[pl.ds(cid * C, C)])
            pltpu.sync_copy(no_v, neg_h.at[pl.ds(cid * CN, CN)])

        first = wid * NCHUNK
        stage(first, 0)

        @pl.loop(0, NCHUNK, step=2)
        def _(g2):
            cid0 = first + g2
            stage(cid0 + 1, 1)
            wait(0)
            compute(cid0, 0)

            @pl.when(g2 + 2 < NCHUNK)
            def _():
                stage(cid0 + 2, 0)

            wait(1)
            compute(cid0 + 1, 1)

    return k(in_pair, ctx_pair, neg_pair, in_par, ctx_par, neg_par,
             tab_in, tab_out)


def _loss_body(pos_ref, neg_ref, o_ref, *, B):
    def ls(x):
        return jnp.minimum(x, 0.0) - jnp.log1p(jnp.exp(-jnp.abs(x)))

    tot = jnp.sum(ls(pos_ref[...])) + jnp.sum(ls(-neg_ref[...]))
    o_ref[0, 0] = -tot / B


@functools.partial(jax.jit, static_argnames=("B",))
def _tc_loss(pos2d, neg2d, *, B):
    return pl.pallas_call(
        functools.partial(_loss_body, B=B),
        out_shape=jax.ShapeDtypeStruct((1, 1), jnp.float32),
        out_specs=pl.BlockSpec(memory_space=pltpu.SMEM),
    )(pos2d, neg2d)


def kernel(inputs, contexts, negatives, in_emb, out_emb):
    B, NEG = negatives.shape
    V, D = in_emb.shape
    in_idx = inputs.reshape(B)
    ctx_idx = contexts.reshape(B)
    neg_idx = negatives.reshape(B * NEG)
    tab_in = in_emb.reshape(V // 2, 2 * D)
    tab_out = out_emb.reshape(V // 2, 2 * D)
    pos, neg = _sc_dots(
        in_idx // 2, ctx_idx // 2, neg_idx // 2,
        in_idx % 2, ctx_idx % 2, neg_idx % 2,
        tab_in, tab_out, B=B, D=D, NEG=NEG)
    loss = _tc_loss(pos.reshape(B // 128, 128), neg.reshape(-1, 128), B=B)
    return loss[0, 0]


# speedup vs baseline: 1.1409x; 1.1409x over previous
import functools

import jax
import jax.numpy as jnp
from jax import lax
from jax.experimental import pallas as pl
from jax.experimental.pallas import tpu as pltpu
from jax.experimental.pallas import tpu_sc as plsc

NC = 2
NS = 16
NW = NC * NS
LANES = 16
C = 16
GI = 32


@functools.partial(jax.jit, static_argnames=("B", "D", "NEG"))
def _sc_dots(in_idx3d, ctx_idx3d, neg_idx3d, tab_in, tab_out, *, B, D, NEG):
    BPW = B // NW
    NCHUNK = BPW // C
    CN = C * NEG
    KD = D // LANES
    D2 = 128
    IBR = BPW // 128
    NBLK = BPW * NEG // 1024

    mesh = plsc.VectorSubcoreMesh(core_axis_name="c", subcore_axis_name="s",
                                  num_cores=NC, num_subcores=NS)

    @functools.partial(
        pl.kernel,
        out_type=(
            jax.ShapeDtypeStruct((NW, 8, 128), jnp.float32),
            jax.ShapeDtypeStruct((NBLK * NW, 8, 128), jnp.float32),
        ),
        mesh=mesh,
        compiler_params=pltpu.CompilerParams(needs_layout_passes=False,
                                             use_tc_tiling_on_sc=True),
        scratch_types=[
            pltpu.VMEM((8, 128), jnp.int32),
            pltpu.VMEM((8, 128), jnp.int32),
            pltpu.VMEM((NBLK, 8, 128), jnp.int32),
            pltpu.VMEM((C,), jnp.int32),
            pltpu.VMEM((C,), jnp.int32),
            pltpu.VMEM((C,), jnp.int32),
            pltpu.VMEM((C,), jnp.int32),
            pltpu.VMEM((CN,), jnp.int32),
            pltpu.VMEM((CN,), jnp.int32),
            pltpu.VMEM((C, D2), jnp.float32),
            pltpu.VMEM((C, D2), jnp.float32),
            pltpu.VMEM((C, D2), jnp.float32),
            pltpu.VMEM((C, D2), jnp.float32),
            pltpu.VMEM((CN, D2), jnp.float32),
            pltpu.VMEM((CN, D2), jnp.float32),
            pltpu.VMEM((8, 128), jnp.float32),
            pltpu.VMEM((NBLK, 8, 128), jnp.float32),
            pltpu.SemaphoreType.DMA,
            pltpu.SemaphoreType.DMA,
        ],
    )
    def k(in_idx_h, ctx_idx_h, neg_idx_h, tab_in_h, tab_out_h,
          pos_h, neg_h,
          irawb, crawb, nrawb, ii0, ii1, ci0, ci1, ni0, ni1,
          ir0, ir1, cr0, cr1, nr0, nr1, po_v, no_v, sem0, sem1):
        ii_v, ci_v, ni_v = (ii0, ii1), (ci0, ci1), (ni0, ni1)
        ir_v, cr_v, nr_v = (ir0, ir1), (cr0, cr1), (nr0, nr1)
        sems = (sem0, sem1)

        wid = lax.axis_index("s") * NC + lax.axis_index("c")
        lane = lax.iota(jnp.int32, LANES)
        lane_masks = [lane == l for l in range(LANES)]

        pltpu.sync_copy(in_idx_h.at[wid], irawb)
        pltpu.sync_copy(ctx_idx_h.at[wid], crawb)
        pltpu.sync_copy(neg_idx_h.at[pl.ds(wid * NBLK, NBLK)], nrawb)

        def raw16(blk, flat):
            if blk is nrawb:
                return blk[flat // 1024, (flat // 128) % 8,
                           pl.ds(flat % 128, LANES)]
            return blk[flat // 128, pl.ds(flat % 128, LANES)]

        def stage(g, b):
            sem = sems[b]
            iraw = raw16(irawb, g * C)
            ii_v[b][...] = iraw
            craw = raw16(crawb, g * C)
            ci_v[b][...] = craw
            for m in range(CN // LANES):
                nraw = raw16(nrawb, g * CN + m * LANES)
                ni_v[b][pl.ds(m * LANES, LANES)] = nraw
            pltpu.make_async_copy(tab_in_h.at[ii_v[b]], ir_v[b], sem).start()
            pltpu.make_async_copy(tab_out_h.at[ci_v[b]], cr_v[b], sem).start()
            for o in range(0, CN, GI):
                pltpu.make_async_copy(
                    tab_out_h.at[ni_v[b].at[pl.ds(o, GI)]],
                    nr_v[b].at[pl.ds(o, GI)], sem).start()

        def wait(b):
            sem = sems[b]
            pltpu.make_async_copy(tab_in_h.at[pl.ds(0, C)],
                                  ir_v[b], sem).wait()
            pltpu.make_async_copy(tab_in_h.at[pl.ds(0, C)],
                                  cr_v[b], sem).wait()
            pltpu.make_async_copy(tab_in_h.at[pl.ds(0, CN)],
                                  nr_v[b], sem).wait()

        def dot(row_chunks, in_chunks):
            acc = in_chunks[0] * row_chunks[0]
            for kk in range(1, KD):
                acc = acc + in_chunks[kk] * row_chunks[kk]
            return jnp.sum(acc)

        def pack16(scalars):
            vec = jnp.zeros((LANES,), jnp.float32)
            for l in range(LANES):
                vec = jnp.where(lane_masks[l], scalars[l], vec)
            return vec

        def compute(g, b):
            scal = []
            for l in range(LANES):
                ivs = [ir_v[b][l, pl.ds(kk * LANES, LANES)]
                       for kk in range(KD)]
                cvs = [cr_v[b][l, pl.ds(kk * LANES, LANES)]
                       for kk in range(KD)]
                scal.append(dot(cvs, ivs))
            flat = g * C
            po_v[flat // 128, pl.ds(flat % 128, LANES)] = pack16(scal)

            @pl.loop(0, CN // LANES)
            def _(v):
                scal = []
                for l in range(LANES):
                    q = v * LANES + l
                    r = q // NEG
                    ivs = [ir_v[b][r, pl.ds(kk * LANES, LANES)]
                           for kk in range(KD)]
                    nvs = [nr_v[b][q, pl.ds(kk * LANES, LANES)]
                           for kk in range(KD)]
                    scal.append(dot(nvs, ivs))
                qf = g * CN + v * LANES
                no_v[qf // 1024, (qf // 128) % 8,
                     pl.ds(qf % 128, LANES)] = pack16(scal)

        stage(0, 0)

        @pl.loop(0, NCHUNK, step=2)
        def _(g2):
            stage(g2 + 1, 1)
            wait(0)
            compute(g2, 0)

            @pl.when(g2 + 2 < NCHUNK)
            def _():
                stage(g2 + 2, 0)

            wait(1)
            compute(g2 + 1, 1)

        pltpu.sync_copy(po_v, pos_h.at[wid])
        pltpu.sync_copy(no_v, neg_h.at[pl.ds(wid * NBLK, NBLK)])

    return k(in_idx3d, ctx_idx3d, neg_idx3d, tab_in, tab_out)


def _loss_body(pos_ref, neg_ref, o_ref, *, B):
    def ls(x):
        return jnp.minimum(x, 0.0) - jnp.log1p(jnp.exp(-jnp.abs(x)))

    tot = jnp.sum(ls(pos_ref[...])) + jnp.sum(ls(-neg_ref[...]))
    o_ref[0, 0] = -tot / B


@functools.partial(jax.jit, static_argnames=("B",))
def _tc_loss(pos2d, neg2d, *, B):
    return pl.pallas_call(
        functools.partial(_loss_body, B=B),
        out_shape=jax.ShapeDtypeStruct((1, 1), jnp.float32),
        out_specs=pl.BlockSpec(memory_space=pltpu.SMEM),
    )(pos2d, neg2d)


def kernel(inputs, contexts, negatives, in_emb, out_emb):
    B, NEG = negatives.shape
    V, D = in_emb.shape
    bpw = B // NW
    ibr = bpw // 128
    in_idx3d = jnp.pad(inputs.reshape(NW, ibr, 128),
                       ((0, 0), (0, 8 - ibr), (0, 0)))
    ctx_idx3d = jnp.pad(contexts.reshape(NW, ibr, 128),
                        ((0, 0), (0, 8 - ibr), (0, 0)))
    neg_idx3d = negatives.reshape(B * NEG // 1024, 8, 128)
    tab_in = jnp.pad(in_emb, ((0, 0), (0, 128 - D)))
    tab_out = jnp.pad(out_emb, ((0, 0), (0, 128 - D)))
    pos3d, neg3d = _sc_dots(in_idx3d, ctx_idx3d, neg_idx3d, tab_in, tab_out,
                            B=B, D=D, NEG=NEG)
    pos = pos3d[:, :ibr, :].reshape(B // 128, 128)
    neg = neg3d.reshape(-1, 128)
    loss = _tc_loss(pos, neg, B=B)
    return loss[0, 0]


# speedup vs baseline: 1.2215x; 1.0707x over previous
import functools

import jax
import jax.numpy as jnp
from jax import lax
from jax.experimental import pallas as pl
from jax.experimental.pallas import tpu as pltpu
from jax.experimental.pallas import tpu_sc as plsc

NC = 2
NS = 16
NW = NC * NS
LANES = 16
C = 16
GI = 32


@functools.partial(jax.jit, static_argnames=("B", "D", "NEG"))
def _sc_dots(in_idx3d, ctx_idx3d, neg_idx3d, tab_in, tab_out, *, B, D, NEG):
    BPW = B // NW
    NCHUNK = BPW // C
    CN = C * NEG
    KD = D // LANES
    D2 = 128
    IBR = BPW // 128
    NBLK = BPW * NEG // 1024

    mesh = plsc.VectorSubcoreMesh(core_axis_name="c", subcore_axis_name="s",
                                  num_cores=NC, num_subcores=NS)

    @functools.partial(
        pl.kernel,
        out_type=(
            jax.ShapeDtypeStruct((NW, 8, 128), jnp.float32),
            jax.ShapeDtypeStruct((NBLK * NW, 8, 128), jnp.float32),
        ),
        mesh=mesh,
        compiler_params=pltpu.CompilerParams(needs_layout_passes=False,
                                             use_tc_tiling_on_sc=True),
        scratch_types=[
            pltpu.VMEM((8, 128), jnp.int32),
            pltpu.VMEM((8, 128), jnp.int32),
            pltpu.VMEM((NBLK, 8, 128), jnp.int32),
            pltpu.VMEM((C,), jnp.int32),
            pltpu.VMEM((C,), jnp.int32),
            pltpu.VMEM((C,), jnp.int32),
            pltpu.VMEM((C,), jnp.int32),
            pltpu.VMEM((CN,), jnp.int32),
            pltpu.VMEM((CN,), jnp.int32),
            pltpu.VMEM((C, D2), jnp.float32),
            pltpu.VMEM((C, D2), jnp.float32),
            pltpu.VMEM((C, D2), jnp.float32),
            pltpu.VMEM((C, D2), jnp.float32),
            pltpu.VMEM((CN, D2), jnp.float32),
            pltpu.VMEM((CN, D2), jnp.float32),
            pltpu.VMEM((8, 128), jnp.float32),
            pltpu.VMEM((NBLK, 8, 128), jnp.float32),
            pltpu.SemaphoreType.DMA,
            pltpu.SemaphoreType.DMA,
        ],
    )
    def k(in_idx_h, ctx_idx_h, neg_idx_h, tab_in_h, tab_out_h,
          pos_h, neg_h,
          irawb, crawb, nrawb, ii0, ii1, ci0, ci1, ni0, ni1,
          ir0, ir1, cr0, cr1, nr0, nr1, po_v, no_v, sem0, sem1):
        ii_v, ci_v, ni_v = (ii0, ii1), (ci0, ci1), (ni0, ni1)
        ir_v, cr_v, nr_v = (ir0, ir1), (cr0, cr1), (nr0, nr1)
        sems = (sem0, sem1)

        wid = lax.axis_index("s") * NC + lax.axis_index("c")
        lane = lax.iota(jnp.int32, LANES)
        lane_masks = [lane == l for l in range(LANES)]

        pltpu.sync_copy(in_idx_h.at[wid], irawb)
        pltpu.sync_copy(ctx_idx_h.at[wid], crawb)
        pltpu.sync_copy(neg_idx_h.at[pl.ds(wid * NBLK, NBLK)], nrawb)

        def raw16(blk, flat):
            if blk is nrawb:
                return blk[flat // 1024, (flat // 128) % 8,
                           pl.ds(flat % 128, LANES)]
            return blk[flat // 128, pl.ds(flat % 128, LANES)]

        def stage(g, b):
            sem = sems[b]
            iraw = raw16(irawb, g * C)
            ii_v[b][...] = iraw
            craw = raw16(crawb, g * C)
            ci_v[b][...] = craw
            for m in range(CN // LANES):
                nraw = raw16(nrawb, g * CN + m * LANES)
                ni_v[b][pl.ds(m * LANES, LANES)] = nraw
            pltpu.make_async_copy(tab_in_h.at[ii_v[b]], ir_v[b], sem).start()
            pltpu.make_async_copy(tab_out_h.at[ci_v[b]], cr_v[b], sem).start()
            for o in range(0, CN, GI):
                pltpu.make_async_copy(
                    tab_out_h.at[ni_v[b].at[pl.ds(o, GI)]],
                    nr_v[b].at[pl.ds(o, GI)], sem).start()

        def wait(b):
            sem = sems[b]
            pltpu.make_async_copy(tab_in_h.at[pl.ds(0, C)],
                                  ir_v[b], sem).wait()
            pltpu.make_async_copy(tab_in_h.at[pl.ds(0, C)],
                                  cr_v[b], sem).wait()
            pltpu.make_async_copy(tab_in_h.at[pl.ds(0, CN)],
                                  nr_v[b], sem).wait()

        def dot(row_chunks, in_chunks):
            acc = in_chunks[0] * row_chunks[0]
            for kk in range(1, KD):
                acc = acc + in_chunks[kk] * row_chunks[kk]
            return jnp.sum(acc)

        def pack16(scalars):
            vec = jnp.zeros((LANES,), jnp.float32)
            for l in range(LANES):
                vec = jnp.where(lane_masks[l], scalars[l], vec)
            return vec

        def compute(g, b):
            scal = []
            for l in range(LANES):
                ivs = [ir_v[b][l, pl.ds(kk * LANES, LANES)]
                       for kk in range(KD)]
                cvs = [cr_v[b][l, pl.ds(kk * LANES, LANES)]
                       for kk in range(KD)]
                scal.append(dot(cvs, ivs))
            flat = g * C
            po_v[flat // 128, pl.ds(flat % 128, LANES)] = pack16(scal)

            @pl.loop(0, CN // LANES)
            def _(v):
                scal = []
                for l in range(LANES):
                    q = v * LANES + l
                    r = q // NEG
                    ivs = [ir_v[b][r, pl.ds(kk * LANES, LANES)]
                           for kk in range(KD)]
                    nvs = [nr_v[b][q, pl.ds(kk * LANES, LANES)]
                           for kk in range(KD)]
                    scal.append(dot(nvs, ivs))
                qf = g * CN + v * LANES
                no_v[qf // 1024, (qf // 128) % 8,
                     pl.ds(qf % 128, LANES)] = pack16(scal)

        stage(0, 0)

        @pl.loop(0, NCHUNK, step=2)
        def _(g2):
            stage(g2 + 1, 1)
            wait(0)
            compute(g2, 0)

            @pl.when(g2 + 2 < NCHUNK)
            def _():
                stage(g2 + 2, 0)

            wait(1)
            compute(g2 + 1, 1)

        pltpu.sync_copy(po_v, pos_h.at[wid])
        pltpu.sync_copy(no_v, neg_h.at[pl.ds(wid * NBLK, NBLK)])

    return k(in_idx3d, ctx_idx3d, neg_idx3d, tab_in, tab_out)


def _pad_body(x_ref, o_ref):
    xt = jnp.transpose(x_ref[...])
    o_ref[...] = jnp.concatenate([xt, jnp.zeros_like(xt)], axis=1)


@jax.jit
def _tc_pad_table(emb_t):
    d, v = emb_t.shape
    cb = 2048
    grid = (v + cb - 1) // cb
    return pl.pallas_call(
        _pad_body,
        grid=(grid,),
        in_specs=[pl.BlockSpec((d, cb), lambda j: (0, j))],
        out_specs=pl.BlockSpec((cb, 2 * d), lambda j: (j, 0)),
        out_shape=jax.ShapeDtypeStruct((v, 2 * d), jnp.float32),
    )(emb_t)


def _loss_body(pos_ref, neg_ref, o_ref, *, B):
    def ls(x):
        return jnp.minimum(x, 0.0) - jnp.log1p(jnp.exp(-jnp.abs(x)))

    tot = jnp.sum(ls(pos_ref[...])) + jnp.sum(ls(-neg_ref[...]))
    o_ref[0, 0] = -tot / B


@functools.partial(jax.jit, static_argnames=("B",))
def _tc_loss(pos2d, neg2d, *, B):
    return pl.pallas_call(
        functools.partial(_loss_body, B=B),
        out_shape=jax.ShapeDtypeStruct((1, 1), jnp.float32),
        out_specs=pl.BlockSpec(memory_space=pltpu.SMEM),
    )(pos2d, neg2d)


def kernel(inputs, contexts, negatives, in_emb, out_emb):
    B, NEG = negatives.shape
    V, D = in_emb.shape
    bpw = B // NW
    ibr = bpw // 128
    in_idx3d = jnp.pad(inputs.reshape(NW, ibr, 128),
                       ((0, 0), (0, 8 - ibr), (0, 0)))
    ctx_idx3d = jnp.pad(contexts.reshape(NW, ibr, 128),
                        ((0, 0), (0, 8 - ibr), (0, 0)))
    neg_idx3d = negatives.reshape(B * NEG // 1024, 8, 128)
    tab_in = _tc_pad_table(in_emb.T)
    tab_out = _tc_pad_table(out_emb.T)
    pos3d, neg3d = _sc_dots(in_idx3d, ctx_idx3d, neg_idx3d, tab_in, tab_out,
                            B=B, D=D, NEG=NEG)
    pos = pos3d[:, :ibr, :].reshape(B // 128, 128)
    neg = neg3d.reshape(-1, 128)
    loss = _tc_loss(pos, neg, B=B)
    return loss[0, 0]


# speedup vs baseline: 1.6820x; 1.3770x over previous
import functools

import jax
import jax.numpy as jnp
from jax import lax
from jax.experimental import pallas as pl
from jax.experimental.pallas import tpu as pltpu
from jax.experimental.pallas import tpu_sc as plsc

NC = 2
NS = 16
NW = NC * NS
LANES = 16
C = 16
GI = 32


@functools.partial(jax.jit, static_argnames=("B", "D", "NEG"))
def _sc_dots(in_idx3d, ctx_idx3d, neg_idx3d, tab_in, tab_out, *, B, D, NEG):
    BPW = B // NW
    NCHUNK = BPW // C
    CN = C * NEG
    KD = D // LANES
    D2 = 128
    IBR = BPW // 128
    NBLK = BPW * NEG // 1024

    mesh = plsc.VectorSubcoreMesh(core_axis_name="c", subcore_axis_name="s",
                                  num_cores=NC, num_subcores=NS)

    @functools.partial(
        pl.kernel,
        out_type=(
            jax.ShapeDtypeStruct((NW, 8, 128), jnp.float32),
            jax.ShapeDtypeStruct((NBLK * NW, 8, 128), jnp.float32),
        ),
        mesh=mesh,
        compiler_params=pltpu.CompilerParams(needs_layout_passes=False,
                                             use_tc_tiling_on_sc=True),
        scratch_types=[
            pltpu.VMEM((8, 128), jnp.int32),
            pltpu.VMEM((8, 128), jnp.int32),
            pltpu.VMEM((NBLK, 8, 128), jnp.int32),
            pltpu.VMEM((C,), jnp.int32),
            pltpu.VMEM((C,), jnp.int32),
            pltpu.VMEM((C,), jnp.int32),
            pltpu.VMEM((C,), jnp.int32),
            pltpu.VMEM((CN,), jnp.int32),
            pltpu.VMEM((CN,), jnp.int32),
            pltpu.VMEM((C, D2), jnp.float32),
            pltpu.VMEM((C, D2), jnp.float32),
            pltpu.VMEM((C, D2), jnp.float32),
            pltpu.VMEM((C, D2), jnp.float32),
            pltpu.VMEM((CN, D2), jnp.float32),
            pltpu.VMEM((CN, D2), jnp.float32),
            pltpu.VMEM((C, D), jnp.float32),
            pltpu.VMEM((8, 128), jnp.float32),
            pltpu.VMEM((NBLK, 8, 128), jnp.float32),
            pltpu.SemaphoreType.DMA,
            pltpu.SemaphoreType.DMA,
        ],
    )
    def k(in_idx_h, ctx_idx_h, neg_idx_h, tab_in_h, tab_out_h,
          pos_h, neg_h,
          irawb, crawb, nrawb, ii0, ii1, ci0, ci1, ni0, ni1,
          ir0, ir1, cr0, cr1, nr0, nr1, is_v, po_v, no_v, sem0, sem1):
        ii_v, ci_v, ni_v = (ii0, ii1), (ci0, ci1), (ni0, ni1)
        ir_v, cr_v, nr_v = (ir0, ir1), (cr0, cr1), (nr0, nr1)
        sems = (sem0, sem1)

        wid = lax.axis_index("s") * NC + lax.axis_index("c")
        lane = lax.iota(jnp.int32, LANES)
        lane_masks = [lane == l for l in range(LANES)]

        pltpu.sync_copy(in_idx_h.at[wid], irawb)
        pltpu.sync_copy(ctx_idx_h.at[wid], crawb)
        pltpu.sync_copy(neg_idx_h.at[pl.ds(wid * NBLK, NBLK)], nrawb)

        def raw16(blk, flat):
            if blk is nrawb:
                return blk[flat // 1024, (flat // 128) % 8,
                           pl.ds(flat % 128, LANES)]
            return blk[flat // 128, pl.ds(flat % 128, LANES)]

        def stage(g, b):
            sem = sems[b]
            iraw = raw16(irawb, g * C)
            ii_v[b][...] = iraw
            craw = raw16(crawb, g * C)
            ci_v[b][...] = craw
            for m in range(CN // LANES):
                nraw = raw16(nrawb, g * CN + m * LANES)
                ni_v[b][pl.ds(m * LANES, LANES)] = nraw
            pltpu.make_async_copy(tab_in_h.at[ii_v[b]], ir_v[b], sem).start()
            pltpu.make_async_copy(tab_out_h.at[ci_v[b]], cr_v[b], sem).start()
            for o in range(0, CN, GI):
                pltpu.make_async_copy(
                    tab_out_h.at[ni_v[b].at[pl.ds(o, GI)]],
                    nr_v[b].at[pl.ds(o, GI)], sem).start()

        def wait(b):
            sem = sems[b]
            pltpu.make_async_copy(tab_in_h.at[pl.ds(0, C)],
                                  ir_v[b], sem).wait()
            pltpu.make_async_copy(tab_in_h.at[pl.ds(0, C)],
                                  cr_v[b], sem).wait()
            pltpu.make_async_copy(tab_in_h.at[pl.ds(0, CN)],
                                  nr_v[b], sem).wait()

        def dot(row_chunks, in_chunks):
            acc = in_chunks[0] * row_chunks[0]
            for kk in range(1, KD):
                acc = acc + in_chunks[kk] * row_chunks[kk]
            return jnp.sum(acc)

        def pack16(scalars):
            vec = jnp.zeros((LANES,), jnp.float32)
            for l in range(LANES):
                vec = jnp.where(lane_masks[l], scalars[l], vec)
            return vec

        def compute(g, b):
            ipar = raw16(irawb, g * C) & 0
            cpar = raw16(crawb, g * C) & 0
            scal = []
            for l in range(LANES):
                ioff = ipar[l] * D
                for kk in range(KD):
                    is_v[l, pl.ds(kk * LANES, LANES)] = (
                        ir_v[b][l, pl.ds(ioff + kk * LANES, LANES)])
                ivs = [is_v[l, pl.ds(kk * LANES, LANES)] for kk in range(KD)]
                coff = cpar[l] * D
                cvs = [cr_v[b][l, pl.ds(coff + kk * LANES, LANES)]
                       for kk in range(KD)]
                scal.append(dot(cvs, ivs))
            flat = g * C
            po_v[flat // 128, pl.ds(flat % 128, LANES)] = pack16(scal)

            @pl.loop(0, CN // LANES)
            def _(v):
                npar = raw16(nrawb, g * CN + v * LANES) & 0
                scal = []
                for l in range(LANES):
                    q = v * LANES + l
                    r = q // NEG
                    noff = npar[l] * D
                    ivs = [is_v[r, pl.ds(kk * LANES, LANES)]
                           for kk in range(KD)]
                    nvs = [nr_v[b][q, pl.ds(noff + kk * LANES, LANES)]
                           for kk in range(KD)]
                    scal.append(dot(nvs, ivs))
                qf = g * CN + v * LANES
                no_v[qf // 1024, (qf // 128) % 8,
                     pl.ds(qf % 128, LANES)] = pack16(scal)

        stage(0, 0)

        @pl.loop(0, NCHUNK, step=2)
        def _(g2):
            stage(g2 + 1, 1)
            wait(0)
            compute(g2, 0)

            @pl.when(g2 + 2 < NCHUNK)
            def _():
                stage(g2 + 2, 0)

            wait(1)
            compute(g2 + 1, 1)

        pltpu.sync_copy(po_v, pos_h.at[wid])
        pltpu.sync_copy(no_v, neg_h.at[pl.ds(wid * NBLK, NBLK)])

    return k(in_idx3d, ctx_idx3d, neg_idx3d, tab_in, tab_out)


def _pair_body(x_ref, o_ref):
    xt = jnp.transpose(x_ref[...])
    o_ref[...] = jnp.concatenate([xt, xt], axis=1)


@jax.jit
def _tc_pair_table(emb_t):
    d, v = emb_t.shape
    cb = 8192
    grid = (v + cb - 1) // cb
    return pl.pallas_call(
        _pair_body,
        grid=(grid,),
        in_specs=[pl.BlockSpec((d, cb), lambda j: (0, j))],
        out_specs=pl.BlockSpec((cb, 2 * d), lambda j: (j, 0)),
        out_shape=jax.ShapeDtypeStruct((v, 2 * d), jnp.float32),
    )(emb_t)


def _loss_body(pos_ref, neg_ref, o_ref, *, B):
    def ls(x):
        return jnp.minimum(x, 0.0) - jnp.log1p(jnp.exp(-jnp.abs(x)))

    tot = jnp.sum(ls(pos_ref[...])) + jnp.sum(ls(-neg_ref[...]))
    o_ref[0, 0] = -tot / B


@functools.partial(jax.jit, static_argnames=("B",))
def _tc_loss(pos2d, neg2d, *, B):
    return pl.pallas_call(
        functools.partial(_loss_body, B=B),
        out_shape=jax.ShapeDtypeStruct((1, 1), jnp.float32),
        out_specs=pl.BlockSpec(memory_space=pltpu.SMEM),
    )(pos2d, neg2d)


def kernel(inputs, contexts, negatives, in_emb, out_emb):
    B, NEG = negatives.shape
    V, D = in_emb.shape
    bpw = B // NW
    ibr = bpw // 128
    in_idx3d = jnp.pad(inputs.reshape(NW, ibr, 128),
                       ((0, 0), (0, 8 - ibr), (0, 0)))
    ctx_idx3d = jnp.pad(contexts.reshape(NW, ibr, 128),
                        ((0, 0), (0, 8 - ibr), (0, 0)))
    neg_idx3d = negatives.reshape(B * NEG // 1024, 8, 128)
    tab_in = _tc_pair_table(in_emb.T)
    tab_out = _tc_pair_table(out_emb.T)
    pos3d, neg3d = _sc_dots(in_idx3d, ctx_idx3d, neg_idx3d, tab_in, tab_out,
                            B=B, D=D, NEG=NEG)
    pos = pos3d[:, :ibr, :].reshape(B // 128, 128)
    neg = neg3d.reshape(-1, 128)
    loss = _tc_loss(pos, neg, B=B)
    return loss[0, 0]


# speedup vs baseline: 1.8482x; 1.0988x over previous
import functools

import jax
import jax.numpy as jnp
from jax import lax
from jax.experimental import pallas as pl
from jax.experimental.pallas import tpu as pltpu
from jax.experimental.pallas import tpu_sc as plsc

NC = 2
NS = 16
NW = NC * NS
LANES = 16
C = 16
GI = 32


@functools.partial(jax.jit, static_argnames=("B", "D", "NEG"))
def _sc_dots(in_idx3d, ctx_idx3d, neg_idx3d, tab_in, tab_out, *, B, D, NEG):
    BPW = B // NW
    NCHUNK = BPW // C
    CN = C * NEG
    KD = D // LANES
    D2 = 128
    IBR = BPW // 128
    NBLK = BPW * NEG // 1024

    mesh = plsc.VectorSubcoreMesh(core_axis_name="c", subcore_axis_name="s",
                                  num_cores=NC, num_subcores=NS)

    @functools.partial(
        pl.kernel,
        out_type=(
            jax.ShapeDtypeStruct((NW, 8, 128), jnp.float32),
            jax.ShapeDtypeStruct((NBLK * NW, 8, 128), jnp.float32),
        ),
        mesh=mesh,
        compiler_params=pltpu.CompilerParams(needs_layout_passes=False,
                                             use_tc_tiling_on_sc=True),
        scratch_types=[
            pltpu.VMEM((8, 128), jnp.int32),
            pltpu.VMEM((8, 128), jnp.int32),
            pltpu.VMEM((NBLK, 8, 128), jnp.int32),
            pltpu.VMEM((C,), jnp.int32),
            pltpu.VMEM((C,), jnp.int32),
            pltpu.VMEM((C,), jnp.int32),
            pltpu.VMEM((C,), jnp.int32),
            pltpu.VMEM((CN,), jnp.int32),
            pltpu.VMEM((CN,), jnp.int32),
            pltpu.VMEM((C, D2), jnp.float32),
            pltpu.VMEM((C, D2), jnp.float32),
            pltpu.VMEM((C, D2), jnp.float32),
            pltpu.VMEM((C, D2), jnp.float32),
            pltpu.VMEM((CN, D2), jnp.float32),
            pltpu.VMEM((CN, D2), jnp.float32),
            pltpu.VMEM((C, D), jnp.float32),
            pltpu.VMEM((8, 128), jnp.float32),
            pltpu.VMEM((NBLK, 8, 128), jnp.float32),
            pltpu.SemaphoreType.DMA,
            pltpu.SemaphoreType.DMA,
        ],
    )
    def k(in_idx_h, ctx_idx_h, neg_idx_h, tab_in_h, tab_out_h,
          pos_h, neg_h,
          irawb, crawb, nrawb, ii0, ii1, ci0, ci1, ni0, ni1,
          ir0, ir1, cr0, cr1, nr0, nr1, is_v, po_v, no_v, sem0, sem1):
        ii_v, ci_v, ni_v = (ii0, ii1), (ci0, ci1), (ni0, ni1)
        ir_v, cr_v, nr_v = (ir0, ir1), (cr0, cr1), (nr0, nr1)
        sems = (sem0, sem1)

        wid = lax.axis_index("s") * NC + lax.axis_index("c")
        lane = lax.iota(jnp.int32, LANES)
        lane_masks = [lane == l for l in range(LANES)]

        pltpu.sync_copy(in_idx_h.at[wid], irawb)
        pltpu.sync_copy(ctx_idx_h.at[wid], crawb)
        pltpu.sync_copy(neg_idx_h.at[pl.ds(wid * NBLK, NBLK)], nrawb)

        def raw16(blk, flat):
            if blk is nrawb:
                return blk[flat // 1024, (flat // 128) % 8,
                           pl.ds(flat % 128, LANES)]
            return blk[flat // 128, pl.ds(flat % 128, LANES)]

        def stage(g, b):
            sem = sems[b]
            iraw = raw16(irawb, g * C)
            ii_v[b][...] = iraw
            craw = raw16(crawb, g * C)
            ci_v[b][...] = craw
            for m in range(CN // LANES):
                nraw = raw16(nrawb, g * CN + m * LANES)
                ni_v[b][pl.ds(m * LANES, LANES)] = nraw
            pltpu.make_async_copy(tab_in_h.at[ii_v[b]], ir_v[b], sem).start()
            pltpu.make_async_copy(tab_out_h.at[ci_v[b]], cr_v[b], sem).start()
            for o in range(0, CN, GI):
                pltpu.make_async_copy(
                    tab_out_h.at[ni_v[b].at[pl.ds(o, GI)]],
                    nr_v[b].at[pl.ds(o, GI)], sem).start()

        def wait(b):
            sem = sems[b]
            pltpu.make_async_copy(tab_in_h.at[pl.ds(0, C)],
                                  ir_v[b], sem).wait()
            pltpu.make_async_copy(tab_in_h.at[pl.ds(0, C)],
                                  cr_v[b], sem).wait()
            pltpu.make_async_copy(tab_in_h.at[pl.ds(0, CN)],
                                  nr_v[b], sem).wait()

        def dot(row_chunks, in_chunks):
            acc = in_chunks[0] * row_chunks[0]
            for kk in range(1, KD):
                acc = acc + in_chunks[kk] * row_chunks[kk]
            return jnp.sum(acc)

        def pack16(scalars):
            vec = jnp.zeros((LANES,), jnp.float32)
            for l in range(LANES):
                vec = jnp.where(lane_masks[l], scalars[l], vec)
            return vec

        def compute(g, b):
            ipar = raw16(irawb, g * C) & 0
            cpar = raw16(crawb, g * C) & 0
            scal = []
            for l in range(LANES):
                ioff = ipar[l] * D
                for kk in range(KD):
                    is_v[l, pl.ds(kk * LANES, LANES)] = (
                        ir_v[b][l, pl.ds(ioff + kk * LANES, LANES)])
                ivs = [is_v[l, pl.ds(kk * LANES, LANES)] for kk in range(KD)]
                coff = cpar[l] * D
                cvs = [cr_v[b][l, pl.ds(coff + kk * LANES, LANES)]
                       for kk in range(KD)]
                scal.append(dot(cvs, ivs))
            flat = g * C
            po_v[flat // 128, pl.ds(flat % 128, LANES)] = pack16(scal)

            @pl.loop(0, CN // LANES)
            def _(v):
                npar = raw16(nrawb, g * CN + v * LANES) & 0
                scal = []
                for l in range(LANES):
                    q = v * LANES + l
                    r = q // NEG
                    noff = npar[l] * D
                    ivs = [is_v[r, pl.ds(kk * LANES, LANES)]
                           for kk in range(KD)]
                    nvs = [nr_v[b][q, pl.ds(noff + kk * LANES, LANES)]
                           for kk in range(KD)]
                    scal.append(dot(nvs, ivs))
                qf = g * CN + v * LANES
                no_v[qf // 1024, (qf // 128) % 8,
                     pl.ds(qf % 128, LANES)] = pack16(scal)

        stage(0, 0)

        @pl.loop(0, NCHUNK, step=2)
        def _(g2):
            stage(g2 + 1, 1)
            wait(0)
            compute(g2, 0)

            @pl.when(g2 + 2 < NCHUNK)
            def _():
                stage(g2 + 2, 0)

            wait(1)
            compute(g2 + 1, 1)

        pltpu.sync_copy(po_v, pos_h.at[wid])
        pltpu.sync_copy(no_v, neg_h.at[pl.ds(wid * NBLK, NBLK)])

    return k(in_idx3d, ctx_idx3d, neg_idx3d, tab_in, tab_out)


def _pair_body(x_ref, o_ref):
    xt = jnp.transpose(x_ref[...])
    o_ref[...] = jnp.concatenate([xt, xt], axis=1)


@jax.jit
def _tc_pair_table(emb_t):
    d, v = emb_t.shape
    cb = 16384
    grid = (v + cb - 1) // cb
    return pl.pallas_call(
        _pair_body,
        grid=(grid,),
        in_specs=[pl.BlockSpec((d, cb), lambda j: (0, j))],
        out_specs=pl.BlockSpec((cb, 2 * d), lambda j: (j, 0)),
        out_shape=jax.ShapeDtypeStruct((v, 2 * d), jnp.float32),
    )(emb_t)


def _loss_body(pos_ref, neg_ref, o_ref, *, B):
    def ls(x):
        return jnp.minimum(x, 0.0) - jnp.log1p(jnp.exp(-jnp.abs(x)))

    tot = jnp.sum(ls(pos_ref[...])) + jnp.sum(ls(-neg_ref[...]))
    o_ref[0, 0] = -tot / B


@functools.partial(jax.jit, static_argnames=("B",))
def _tc_loss(pos2d, neg2d, *, B):
    return pl.pallas_call(
        functools.partial(_loss_body, B=B),
        out_shape=jax.ShapeDtypeStruct((1, 1), jnp.float32),
        out_specs=pl.BlockSpec(memory_space=pltpu.SMEM),
    )(pos2d, neg2d)


def kernel(inputs, contexts, negatives, in_emb, out_emb):
    B, NEG = negatives.shape
    V, D = in_emb.shape
    bpw = B // NW
    ibr = bpw // 128
    in_idx3d = jnp.pad(inputs.reshape(NW, ibr, 128),
                       ((0, 0), (0, 8 - ibr), (0, 0)))
    ctx_idx3d = jnp.pad(contexts.reshape(NW, ibr, 128),
                        ((0, 0), (0, 8 - ibr), (0, 0)))
    neg_idx3d = negatives.reshape(B * NEG // 1024, 8, 128)
    tab_in = _tc_pair_table(in_emb.T)
    tab_out = _tc_pair_table(out_emb.T)
    pos3d, neg3d = _sc_dots(in_idx3d, ctx_idx3d, neg_idx3d, tab_in, tab_out,
                            B=B, D=D, NEG=NEG)
    pos = pos3d[:, :ibr, :].reshape(B // 128, 128)
    neg = neg3d.reshape(-1, 128)
    loss = _tc_loss(pos, neg, B=B)
    return loss[0, 0]


# speedup vs baseline: 1.9047x; 1.0306x over previous
import functools

import jax
import jax.numpy as jnp
from jax import lax
from jax.experimental import pallas as pl
from jax.experimental.pallas import tpu as pltpu
from jax.experimental.pallas import tpu_sc as plsc

NC = 2
NS = 16
NW = NC * NS
LANES = 16
C = 16
GI = 32


@functools.partial(jax.jit, static_argnames=("B", "D", "NEG"))
def _sc_dots(in_idx3d, ctx_idx3d, neg_idx3d, tab_in, tab_out, *, B, D, NEG):
    BPW = B // NW
    NCHUNK = BPW // C
    CN = C * NEG
    KD = D // LANES
    D2 = 128
    IBR = BPW // 128
    NBLK = BPW * NEG // 1024

    mesh = plsc.VectorSubcoreMesh(core_axis_name="c", subcore_axis_name="s",
                                  num_cores=NC, num_subcores=NS)

    @functools.partial(
        pl.kernel,
        out_type=(
            jax.ShapeDtypeStruct((NW, 8, 128), jnp.float32),
            jax.ShapeDtypeStruct((NBLK * NW, 8, 128), jnp.float32),
        ),
        mesh=mesh,
        compiler_params=pltpu.CompilerParams(needs_layout_passes=False,
                                             use_tc_tiling_on_sc=True),
        scratch_types=[
            pltpu.VMEM((8, 128), jnp.int32),
            pltpu.VMEM((8, 128), jnp.int32),
            pltpu.VMEM((NBLK, 8, 128), jnp.int32),
            pltpu.VMEM((C,), jnp.int32),
            pltpu.VMEM((C,), jnp.int32),
            pltpu.VMEM((C,), jnp.int32),
            pltpu.VMEM((C,), jnp.int32),
            pltpu.VMEM((CN,), jnp.int32),
            pltpu.VMEM((CN,), jnp.int32),
            pltpu.VMEM((C, D2), jnp.float32),
            pltpu.VMEM((C, D2), jnp.float32),
            pltpu.VMEM((C, D2), jnp.float32),
            pltpu.VMEM((C, D2), jnp.float32),
            pltpu.VMEM((CN, D2), jnp.float32),
            pltpu.VMEM((CN, D2), jnp.float32),
            pltpu.VMEM((C, D), jnp.float32),
            pltpu.VMEM((8, 128), jnp.float32),
            pltpu.VMEM((NBLK, 8, 128), jnp.float32),
            pltpu.SemaphoreType.DMA,
            pltpu.SemaphoreType.DMA,
        ],
    )
    def k(in_idx_h, ctx_idx_h, neg_idx_h, tab_in_h, tab_out_h,
          pos_h, neg_h,
          irawb, crawb, nrawb, ii0, ii1, ci0, ci1, ni0, ni1,
          ir0, ir1, cr0, cr1, nr0, nr1, is_v, po_v, no_v, sem0, sem1):
        ii_v, ci_v, ni_v = (ii0, ii1), (ci0, ci1), (ni0, ni1)
        ir_v, cr_v, nr_v = (ir0, ir1), (cr0, cr1), (nr0, nr1)
        sems = (sem0, sem1)

        wid = lax.axis_index("s") * NC + lax.axis_index("c")
        lane = lax.iota(jnp.int32, LANES)
        lane_masks = [lane == l for l in range(LANES)]

        pltpu.sync_copy(in_idx_h.at[wid], irawb)
        pltpu.sync_copy(ctx_idx_h.at[wid], crawb)
        pltpu.sync_copy(neg_idx_h.at[pl.ds(wid * NBLK, NBLK)], nrawb)

        def raw16(blk, flat):
            if blk is nrawb:
                return blk[flat // 1024, (flat // 128) % 8,
                           pl.ds(flat % 128, LANES)]
            return blk[flat // 128, pl.ds(flat % 128, LANES)]

        def stage(g, b):
            sem = sems[b]
            iraw = raw16(irawb, g * C)
            ii_v[b][...] = iraw
            craw = raw16(crawb, g * C)
            ci_v[b][...] = craw
            for m in range(CN // LANES):
                nraw = raw16(nrawb, g * CN + m * LANES)
                ni_v[b][pl.ds(m * LANES, LANES)] = nraw
            pltpu.make_async_copy(tab_in_h.at[ii_v[b]], ir_v[b], sem).start()
            pltpu.make_async_copy(tab_out_h.at[ci_v[b]], cr_v[b], sem).start()
            for o in range(0, CN, GI):
                pltpu.make_async_copy(
                    tab_out_h.at[ni_v[b].at[pl.ds(o, GI)]],
                    nr_v[b].at[pl.ds(o, GI)], sem).start()

        def wait(b):
            sem = sems[b]
            pltpu.make_async_copy(tab_in_h.at[pl.ds(0, C)],
                                  ir_v[b], sem).wait()
            pltpu.make_async_copy(tab_in_h.at[pl.ds(0, C)],
                                  cr_v[b], sem).wait()
            pltpu.make_async_copy(tab_in_h.at[pl.ds(0, CN)],
                                  nr_v[b], sem).wait()

        def dot(row_chunks, in_chunks):
            acc = in_chunks[0] * row_chunks[0]
            for kk in range(1, KD):
                acc = acc + in_chunks[kk] * row_chunks[kk]
            return jnp.sum(acc)

        def pack16(scalars):
            vec = jnp.zeros((LANES,), jnp.float32)
            for l in range(LANES):
                vec = jnp.where(lane_masks[l], scalars[l], vec)
            return vec

        def compute(g, b):
            ipar = raw16(irawb, g * C) & 0
            cpar = raw16(crawb, g * C) & 0
            scal = []
            for l in range(LANES):
                ioff = ipar[l] * D
                for kk in range(KD):
                    is_v[l, pl.ds(kk * LANES, LANES)] = (
                        ir_v[b][l, pl.ds(ioff + kk * LANES, LANES)])
                ivs = [is_v[l, pl.ds(kk * LANES, LANES)] for kk in range(KD)]
                coff = cpar[l] * D
                cvs = [cr_v[b][l, pl.ds(coff + kk * LANES, LANES)]
                       for kk in range(KD)]
                scal.append(dot(cvs, ivs))
            flat = g * C
            po_v[flat // 128, pl.ds(flat % 128, LANES)] = pack16(scal)

            @pl.loop(0, CN // LANES)
            def _(v):
                npar = raw16(nrawb, g * CN + v * LANES) & 0
                scal = []
                for l in range(LANES):
                    q = v * LANES + l
                    r = q // NEG
                    noff = npar[l] * D
                    ivs = [is_v[r, pl.ds(kk * LANES, LANES)]
                           for kk in range(KD)]
                    nvs = [nr_v[b][q, pl.ds(noff + kk * LANES, LANES)]
                           for kk in range(KD)]
                    scal.append(dot(nvs, ivs))
                qf = g * CN + v * LANES
                no_v[qf // 1024, (qf // 128) % 8,
                     pl.ds(qf % 128, LANES)] = pack16(scal)

        stage(0, 0)

        @pl.loop(0, NCHUNK, step=2)
        def _(g2):
            stage(g2 + 1, 1)
            wait(0)
            compute(g2, 0)

            @pl.when(g2 + 2 < NCHUNK)
            def _():
                stage(g2 + 2, 0)

            wait(1)
            compute(g2 + 1, 1)

        pltpu.sync_copy(po_v, pos_h.at[wid])
        pltpu.sync_copy(no_v, neg_h.at[pl.ds(wid * NBLK, NBLK)])

    return k(in_idx3d, ctx_idx3d, neg_idx3d, tab_in, tab_out)


def _pair_body(x_ref, o_ref):
    xt = jnp.transpose(x_ref[...])
    o_ref[...] = jnp.concatenate([xt, xt], axis=1)


@jax.jit
def _tc_pair_table(emb_t):
    d, v = emb_t.shape
    cb = 24576
    grid = (v + cb - 1) // cb
    return pl.pallas_call(
        _pair_body,
        grid=(grid,),
        in_specs=[pl.BlockSpec((d, cb), lambda j: (0, j))],
        out_specs=pl.BlockSpec((cb, 2 * d), lambda j: (j, 0)),
        out_shape=jax.ShapeDtypeStruct((v, 2 * d), jnp.float32),
    )(emb_t)


def _loss_body(pos_ref, neg_ref, o_ref, *, B):
    def ls(x):
        return jnp.minimum(x, 0.0) - jnp.log1p(jnp.exp(-jnp.abs(x)))

    tot = jnp.sum(ls(pos_ref[...])) + jnp.sum(ls(-neg_ref[...]))
    o_ref[0, 0] = -tot / B


@functools.partial(jax.jit, static_argnames=("B",))
def _tc_loss(pos2d, neg2d, *, B):
    return pl.pallas_call(
        functools.partial(_loss_body, B=B),
        out_shape=jax.ShapeDtypeStruct((1, 1), jnp.float32),
        out_specs=pl.BlockSpec(memory_space=pltpu.SMEM),
    )(pos2d, neg2d)


def kernel(inputs, contexts, negatives, in_emb, out_emb):
    B, NEG = negatives.shape
    V, D = in_emb.shape
    bpw = B // NW
    ibr = bpw // 128
    in_idx3d = jnp.pad(inputs.reshape(NW, ibr, 128),
                       ((0, 0), (0, 8 - ibr), (0, 0)))
    ctx_idx3d = jnp.pad(contexts.reshape(NW, ibr, 128),
                        ((0, 0), (0, 8 - ibr), (0, 0)))
    neg_idx3d = negatives.reshape(B * NEG // 1024, 8, 128)
    tab_in = _tc_pair_table(in_emb.T)
    tab_out = _tc_pair_table(out_emb.T)
    pos3d, neg3d = _sc_dots(in_idx3d, ctx_idx3d, neg_idx3d, tab_in, tab_out,
                            B=B, D=D, NEG=NEG)
    pos = pos3d[:, :ibr, :].reshape(B // 128, 128)
    neg = neg3d.reshape(-1, 128)
    loss = _tc_loss(pos, neg, B=B)
    return loss[0, 0]


# speedup vs baseline: 1.9068x; 1.0011x over previous
import functools

import jax
import jax.numpy as jnp
from jax import lax
from jax.experimental import pallas as pl
from jax.experimental.pallas import tpu as pltpu
from jax.experimental.pallas import tpu_sc as plsc

NC = 2
NS = 16
NW = NC * NS
LANES = 16
C = 16
GI = 32


@functools.partial(jax.jit, static_argnames=("B", "D", "NEG"))
def _sc_dots(in_idx3d, ctx_idx3d, neg_idx3d, tab_in, tab_out, *, B, D, NEG):
    BPW = B // NW
    NCHUNK = BPW // C
    CN = C * NEG
    KD = D // LANES
    D2 = 128
    IBR = BPW // 128
    NBLK = BPW * NEG // 1024

    mesh = plsc.VectorSubcoreMesh(core_axis_name="c", subcore_axis_name="s",
                                  num_cores=NC, num_subcores=NS)

    @functools.partial(
        pl.kernel,
        out_type=(
            jax.ShapeDtypeStruct((NW, 8, 128), jnp.float32),
            jax.ShapeDtypeStruct((NBLK * NW, 8, 128), jnp.float32),
        ),
        mesh=mesh,
        compiler_params=pltpu.CompilerParams(needs_layout_passes=False,
                                             use_tc_tiling_on_sc=True),
        scratch_types=[
            pltpu.VMEM((8, 128), jnp.int32),
            pltpu.VMEM((8, 128), jnp.int32),
            pltpu.VMEM((NBLK, 8, 128), jnp.int32),
            pltpu.VMEM((C,), jnp.int32),
            pltpu.VMEM((C,), jnp.int32),
            pltpu.VMEM((C,), jnp.int32),
            pltpu.VMEM((C,), jnp.int32),
            pltpu.VMEM((CN,), jnp.int32),
            pltpu.VMEM((CN,), jnp.int32),
            pltpu.VMEM((C, D2), jnp.float32),
            pltpu.VMEM((C, D2), jnp.float32),
            pltpu.VMEM((C, D2), jnp.float32),
            pltpu.VMEM((C, D2), jnp.float32),
            pltpu.VMEM((CN, D2), jnp.float32),
            pltpu.VMEM((CN, D2), jnp.float32),
            pltpu.VMEM((8, 128), jnp.float32),
            pltpu.VMEM((NBLK, 8, 128), jnp.float32),
            pltpu.SemaphoreType.DMA,
            pltpu.SemaphoreType.DMA,
        ],
    )
    def k(in_idx_h, ctx_idx_h, neg_idx_h, tab_in_h, tab_out_h,
          pos_h, neg_h,
          irawb, crawb, nrawb, ii0, ii1, ci0, ci1, ni0, ni1,
          ir0, ir1, cr0, cr1, nr0, nr1, po_v, no_v, sem0, sem1):
        ii_v, ci_v, ni_v = (ii0, ii1), (ci0, ci1), (ni0, ni1)
        ir_v, cr_v, nr_v = (ir0, ir1), (cr0, cr1), (nr0, nr1)
        sems = (sem0, sem1)

        wid = lax.axis_index("s") * NC + lax.axis_index("c")
        lane = lax.iota(jnp.int32, LANES)
        lane_masks = [lane == l for l in range(LANES)]

        pltpu.sync_copy(in_idx_h.at[wid], irawb)
        pltpu.sync_copy(ctx_idx_h.at[wid], crawb)
        pltpu.sync_copy(neg_idx_h.at[pl.ds(wid * NBLK, NBLK)], nrawb)

        def raw16(blk, flat):
            if blk is nrawb:
                return blk[flat // 1024, (flat // 128) % 8,
                           pl.ds(flat % 128, LANES)]
            return blk[flat // 128, pl.ds(flat % 128, LANES)]

        def stage(g, b):
            sem = sems[b]
            iraw = raw16(irawb, g * C)
            ii_v[b][...] = iraw
            craw = raw16(crawb, g * C)
            ci_v[b][...] = craw
            for m in range(CN // LANES):
                nraw = raw16(nrawb, g * CN + m * LANES)
                ni_v[b][pl.ds(m * LANES, LANES)] = nraw
            pltpu.make_async_copy(tab_in_h.at[ii_v[b]], ir_v[b], sem).start()
            pltpu.make_async_copy(tab_out_h.at[ci_v[b]], cr_v[b], sem).start()
            for o in range(0, CN, GI):
                pltpu.make_async_copy(
                    tab_out_h.at[ni_v[b].at[pl.ds(o, GI)]],
                    nr_v[b].at[pl.ds(o, GI)], sem).start()

        def wait(b):
            sem = sems[b]
            pltpu.make_async_copy(tab_in_h.at[pl.ds(0, C)],
                                  ir_v[b], sem).wait()
            pltpu.make_async_copy(tab_in_h.at[pl.ds(0, C)],
                                  cr_v[b], sem).wait()
            pltpu.make_async_copy(tab_in_h.at[pl.ds(0, CN)],
                                  nr_v[b], sem).wait()

        def dot(row_chunks, in_chunks):
            acc = in_chunks[0] * row_chunks[0]
            for kk in range(1, KD):
                acc = acc + in_chunks[kk] * row_chunks[kk]
            return jnp.sum(acc)

        def pack16(scalars):
            vec = jnp.zeros((LANES,), jnp.float32)
            for l in range(LANES):
                vec = jnp.where(lane_masks[l], scalars[l], vec)
            return vec

        def compute(g, b):
            scal = []
            for l in range(LANES):
                ivs = [ir_v[b][l, pl.ds(kk * LANES, LANES)]
                       for kk in range(KD)]
                cvs = [cr_v[b][l, pl.ds(kk * LANES, LANES)]
                       for kk in range(KD)]
                scal.append(dot(cvs, ivs))
            flat = g * C
            po_v[flat // 128, pl.ds(flat % 128, LANES)] = pack16(scal)

            @pl.loop(0, CN // LANES)
            def _(v):
                scal = []
                for l in range(LANES):
                    q = v * LANES + l
                    r = q // NEG
                    ivs = [ir_v[b][r, pl.ds(kk * LANES, LANES)]
                           for kk in range(KD)]
                    nvs = [nr_v[b][q, pl.ds(kk * LANES, LANES)]
                           for kk in range(KD)]
                    scal.append(dot(nvs, ivs))
                qf = g * CN + v * LANES
                no_v[qf // 1024, (qf // 128) % 8,
                     pl.ds(qf % 128, LANES)] = pack16(scal)

        stage(0, 0)

        @pl.loop(0, NCHUNK, step=2)
        def _(g2):
            stage(g2 + 1, 1)
            wait(0)
            compute(g2, 0)

            @pl.when(g2 + 2 < NCHUNK)
            def _():
                stage(g2 + 2, 0)

            wait(1)
            compute(g2 + 1, 1)

        pltpu.sync_copy(po_v, pos_h.at[wid])
        pltpu.sync_copy(no_v, neg_h.at[pl.ds(wid * NBLK, NBLK)])

    return k(in_idx3d, ctx_idx3d, neg_idx3d, tab_in, tab_out)


def _pair_body(x_ref, o_ref):
    xt = jnp.transpose(x_ref[...])
    o_ref[...] = jnp.concatenate([xt, xt], axis=1)


@jax.jit
def _tc_pair_table(emb_t):
    d, v = emb_t.shape
    cb = 24576
    grid = (v + cb - 1) // cb
    return pl.pallas_call(
        _pair_body,
        grid=(grid,),
        in_specs=[pl.BlockSpec((d, cb), lambda j: (0, j))],
        out_specs=pl.BlockSpec((cb, 2 * d), lambda j: (j, 0)),
        out_shape=jax.ShapeDtypeStruct((v, 2 * d), jnp.float32),
    )(emb_t)


def _loss_body(pos_ref, neg_ref, o_ref, *, B):
    def ls(x):
        return jnp.minimum(x, 0.0) - jnp.log1p(jnp.exp(-jnp.abs(x)))

    tot = jnp.sum(ls(pos_ref[...])) + jnp.sum(ls(-neg_ref[...]))
    o_ref[0, 0] = -tot / B


@functools.partial(jax.jit, static_argnames=("B",))
def _tc_loss(pos2d, neg2d, *, B):
    return pl.pallas_call(
        functools.partial(_loss_body, B=B),
        out_shape=jax.ShapeDtypeStruct((1, 1), jnp.float32),
        out_specs=pl.BlockSpec(memory_space=pltpu.SMEM),
    )(pos2d, neg2d)


def kernel(inputs, contexts, negatives, in_emb, out_emb):
    B, NEG = negatives.shape
    V, D = in_emb.shape
    bpw = B // NW
    ibr = bpw // 128
    in_idx3d = jnp.pad(inputs.reshape(NW, ibr, 128),
                       ((0, 0), (0, 8 - ibr), (0, 0)))
    ctx_idx3d = jnp.pad(contexts.reshape(NW, ibr, 128),
                        ((0, 0), (0, 8 - ibr), (0, 0)))
    neg_idx3d = negatives.reshape(B * NEG // 1024, 8, 128)
    tab_in = _tc_pair_table(in_emb.T)
    tab_out = _tc_pair_table(out_emb.T)
    pos3d, neg3d = _sc_dots(in_idx3d, ctx_idx3d, neg_idx3d, tab_in, tab_out,
                            B=B, D=D, NEG=NEG)
    pos = pos3d[:, :ibr, :].reshape(B // 128, 128)
    neg = neg3d.reshape(-1, 128)
    loss = _tc_loss(pos, neg, B=B)
    return loss[0, 0]


# speedup vs baseline: 2.4138x; 1.2659x over previous
import functools

import jax
import jax.numpy as jnp
from jax import lax
from jax.experimental import pallas as pl
from jax.experimental.pallas import tpu as pltpu
from jax.experimental.pallas import tpu_sc as plsc

NC = 2
NS = 16
NW = NC * NS
LANES = 16
C = 16
GI = 32


@functools.partial(jax.jit, static_argnames=("B", "D", "NEG"))
def _sc_dots(in_idx3d, ctx_idx3d, neg_idx3d, tab, *, B, D, NEG):
    BPW = B // NW
    NCHUNK = BPW // C
    CN = C * NEG
    KD = D // LANES
    D2 = 128
    IBR = BPW // 128
    NBLK = BPW * NEG // 1024

    mesh = plsc.VectorSubcoreMesh(core_axis_name="c", subcore_axis_name="s",
                                  num_cores=NC, num_subcores=NS)

    @functools.partial(
        pl.kernel,
        out_type=(
            jax.ShapeDtypeStruct((NW, 8, 128), jnp.float32),
            jax.ShapeDtypeStruct((NBLK * NW, 8, 128), jnp.float32),
        ),
        mesh=mesh,
        compiler_params=pltpu.CompilerParams(needs_layout_passes=False,
                                             use_tc_tiling_on_sc=True),
        scratch_types=[
            pltpu.VMEM((8, 128), jnp.int32),
            pltpu.VMEM((8, 128), jnp.int32),
            pltpu.VMEM((NBLK, 8, 128), jnp.int32),
            pltpu.VMEM((C,), jnp.int32),
            pltpu.VMEM((C,), jnp.int32),
            pltpu.VMEM((C,), jnp.int32),
            pltpu.VMEM((C,), jnp.int32),
            pltpu.VMEM((CN,), jnp.int32),
            pltpu.VMEM((CN,), jnp.int32),
            pltpu.VMEM((C, D2), jnp.float32),
            pltpu.VMEM((C, D2), jnp.float32),
            pltpu.VMEM((C, D2), jnp.float32),
            pltpu.VMEM((C, D2), jnp.float32),
            pltpu.VMEM((CN, D2), jnp.float32),
            pltpu.VMEM((CN, D2), jnp.float32),
            pltpu.VMEM((8, 128), jnp.float32),
            pltpu.VMEM((NBLK, 8, 128), jnp.float32),
            pltpu.SemaphoreType.DMA,
            pltpu.SemaphoreType.DMA,
        ],
    )
    def k(in_idx_h, ctx_idx_h, neg_idx_h, tab_h,
          pos_h, neg_h,
          irawb, crawb, nrawb, ii0, ii1, ci0, ci1, ni0, ni1,
          ir0, ir1, cr0, cr1, nr0, nr1, po_v, no_v, sem0, sem1):
        ii_v, ci_v, ni_v = (ii0, ii1), (ci0, ci1), (ni0, ni1)
        ir_v, cr_v, nr_v = (ir0, ir1), (cr0, cr1), (nr0, nr1)
        sems = (sem0, sem1)

        wid = lax.axis_index("s") * NC + lax.axis_index("c")
        lane = lax.iota(jnp.int32, LANES)
        lane_masks = [lane == l for l in range(LANES)]

        pltpu.sync_copy(in_idx_h.at[wid], irawb)
        pltpu.sync_copy(ctx_idx_h.at[wid], crawb)
        pltpu.sync_copy(neg_idx_h.at[pl.ds(wid * NBLK, NBLK)], nrawb)

        def raw16(blk, flat):
            if blk is nrawb:
                return blk[flat // 1024, (flat // 128) % 8,
                           pl.ds(flat % 128, LANES)]
            return blk[flat // 128, pl.ds(flat % 128, LANES)]

        def stage(g, b):
            sem = sems[b]
            iraw = raw16(irawb, g * C)
            ii_v[b][...] = iraw
            craw = raw16(crawb, g * C)
            ci_v[b][...] = craw
            for m in range(CN // LANES):
                nraw = raw16(nrawb, g * CN + m * LANES)
                ni_v[b][pl.ds(m * LANES, LANES)] = nraw
            pltpu.make_async_copy(tab_h.at[ii_v[b]], ir_v[b], sem).start()
            pltpu.make_async_copy(tab_h.at[ci_v[b]], cr_v[b], sem).start()
            for o in range(0, CN, GI):
                pltpu.make_async_copy(
                    tab_h.at[ni_v[b].at[pl.ds(o, GI)]],
                    nr_v[b].at[pl.ds(o, GI)], sem).start()

        def wait(b):
            sem = sems[b]
            pltpu.make_async_copy(tab_h.at[pl.ds(0, C)],
                                  ir_v[b], sem).wait()
            pltpu.make_async_copy(tab_h.at[pl.ds(0, C)],
                                  cr_v[b], sem).wait()
            pltpu.make_async_copy(tab_h.at[pl.ds(0, CN)],
                                  nr_v[b], sem).wait()

        def dot(row_chunks, in_chunks):
            acc = in_chunks[0] * row_chunks[0]
            for kk in range(1, KD):
                acc = acc + in_chunks[kk] * row_chunks[kk]
            return jnp.sum(acc)

        def pack16(scalars):
            vec = jnp.zeros((LANES,), jnp.float32)
            for l in range(LANES):
                vec = jnp.where(lane_masks[l], scalars[l], vec)
            return vec

        def compute(g, b):
            scal = []
            for l in range(LANES):
                ivs = [ir_v[b][l, pl.ds(kk * LANES, LANES)]
                       for kk in range(KD)]
                cvs = [cr_v[b][l, pl.ds(D + kk * LANES, LANES)]
                       for kk in range(KD)]
                scal.append(dot(cvs, ivs))
            flat = g * C
            po_v[flat // 128, pl.ds(flat % 128, LANES)] = pack16(scal)

            @pl.loop(0, CN // LANES)
            def _(v):
                scal = []
                for l in range(LANES):
                    q = v * LANES + l
                    r = q // NEG
                    ivs = [ir_v[b][r, pl.ds(kk * LANES, LANES)]
                           for kk in range(KD)]
                    nvs = [nr_v[b][q, pl.ds(D + kk * LANES, LANES)]
                           for kk in range(KD)]
                    scal.append(dot(nvs, ivs))
                qf = g * CN + v * LANES
                no_v[qf // 1024, (qf // 128) % 8,
                     pl.ds(qf % 128, LANES)] = pack16(scal)

        stage(0, 0)

        @pl.loop(0, NCHUNK, step=2)
        def _(g2):
            stage(g2 + 1, 1)
            wait(0)
            compute(g2, 0)

            @pl.when(g2 + 2 < NCHUNK)
            def _():
                stage(g2 + 2, 0)

            wait(1)
            compute(g2 + 1, 1)

        pltpu.sync_copy(po_v, pos_h.at[wid])
        pltpu.sync_copy(no_v, neg_h.at[pl.ds(wid * NBLK, NBLK)])

    return k(in_idx3d, ctx_idx3d, neg_idx3d, tab)


def _merge_body(x_ref, y_ref, o_ref):
    o_ref[...] = jnp.concatenate(
        [jnp.transpose(x_ref[...]), jnp.transpose(y_ref[...])], axis=1)


@jax.jit
def _tc_merge_tables(in_t, out_t):
    d, v = in_t.shape
    cb = 16384
    grid = (v + cb - 1) // cb
    return pl.pallas_call(
        _merge_body,
        grid=(grid,),
        in_specs=[pl.BlockSpec((d, cb), lambda j: (0, j)),
                  pl.BlockSpec((d, cb), lambda j: (0, j))],
        out_specs=pl.BlockSpec((cb, 2 * d), lambda j: (j, 0)),
        out_shape=jax.ShapeDtypeStruct((v, 2 * d), jnp.float32),
    )(in_t, out_t)


def _loss_body(pos_ref, neg_ref, o_ref, *, B):
    def ls(x):
        return jnp.minimum(x, 0.0) - jnp.log1p(jnp.exp(-jnp.abs(x)))

    tot = jnp.sum(ls(pos_ref[...])) + jnp.sum(ls(-neg_ref[...]))
    o_ref[0, 0] = -tot / B


@functools.partial(jax.jit, static_argnames=("B",))
def _tc_loss(pos2d, neg2d, *, B):
    return pl.pallas_call(
        functools.partial(_loss_body, B=B),
        out_shape=jax.ShapeDtypeStruct((1, 1), jnp.float32),
        out_specs=pl.BlockSpec(memory_space=pltpu.SMEM),
    )(pos2d, neg2d)


def kernel(inputs, contexts, negatives, in_emb, out_emb):
    B, NEG = negatives.shape
    V, D = in_emb.shape
    bpw = B // NW
    ibr = bpw // 128
    in_idx3d = jnp.pad(inputs.reshape(NW, ibr, 128),
                       ((0, 0), (0, 8 - ibr), (0, 0)))
    ctx_idx3d = jnp.pad(contexts.reshape(NW, ibr, 128),
                        ((0, 0), (0, 8 - ibr), (0, 0)))
    neg_idx3d = negatives.reshape(B * NEG // 1024, 8, 128)
    tab = _tc_merge_tables(in_emb.T, out_emb.T)
    pos3d, neg3d = _sc_dots(in_idx3d, ctx_idx3d, neg_idx3d, tab,
                            B=B, D=D, NEG=NEG)
    pos = pos3d[:, :ibr, :].reshape(B // 128, 128)
    neg = neg3d.reshape(-1, 128)
    loss = _tc_loss(pos, neg, B=B)
    return loss[0, 0]
